# Initial kernel scaffold; baseline (speedup 1.0000x reference)
import jax
import jax.numpy as jnp
from jax.experimental import pallas as pl


def kernel(words, attr_emb, word_emb_table, a):
    raise NotImplementedError("write your pallas kernel here")


# speedup vs baseline: 10.2237x; 10.2237x over previous
import functools

import jax
import jax.numpy as jnp
from jax import lax
from jax.experimental import pallas as pl
from jax.experimental.pallas import tpu as pltpu
from jax.experimental.pallas import tpu_sc as plsc

B, L, V, D = 1024, 50, 100000, 128
LP = 64
NW = 32
RPW = B // NW
VCH = 2500
NCH = 4


def _tc_body(tab_ref, a1_ref, attr_ref, a2_ref, t_ref, c_ref):
    t_ref[...] = jnp.sum(tab_ref[...] * a1_ref[...][None], axis=-1)
    @pl.when(pl.program_id(0) == 0)
    def _():
        c_ref[...] = jnp.sum(attr_ref[...] * a2_ref[...], axis=-1)


def _sc_body(t_hbm, c_hbm, w_hbm, out_hbm, wv, tv, cv, rowbuf, gsem):
    wid = lax.axis_index("s") * 2 + lax.axis_index("c")
    base = wid * RPW

    pltpu.sync_copy(w_hbm.at[pl.ds(base, RPW)], wv)
    pltpu.sync_copy(c_hbm.at[pl.ds(base, RPW)], cv)

    descs = [
        pltpu.async_copy(t_hbm.at[wv.at[r]], tv.at[r], gsem)
        for r in range(RPW)
    ]
    for d in descs:
        d.wait()

    @pl.loop(0, V // 16)
    def _zero(i):
        rowbuf[pl.ds(i * 16, 16)] = jnp.zeros((16,), jnp.float32)

    lane = lax.iota(jnp.int32, 16)
    cvecs = [cv[pl.ds(k * 16, 16)] for k in range(RPW // 16)]
    for r in range(RPW):
        cr = cvecs[r // 16][r % 16]
        for ch in range(NCH):
            w16 = wv[r, pl.ds(ch * 16, 16)]
            t16 = tv[r, pl.ds(ch * 16, 16)]
            x = t16 + cr
            e16 = jnp.maximum(x, 0.2 * x)
            if (ch + 1) * 16 <= L:
                plsc.store_scatter(rowbuf, [w16], e16)
            else:
                plsc.store_scatter(rowbuf, [w16], e16, mask=lane < (L - ch * 16))
        pltpu.sync_copy(rowbuf, out_hbm.at[base + r])
        z16 = jnp.zeros((16,), jnp.float32)
        for ch in range(NCH):
            w16 = wv[r, pl.ds(ch * 16, 16)]
            if (ch + 1) * 16 <= L:
                plsc.store_scatter(rowbuf, [w16], z16)
            else:
                plsc.store_scatter(rowbuf, [w16], z16, mask=lane < (L - ch * 16))


def kernel(words, attr_emb, word_emb_table, a):
    a1 = a[:D, 0].reshape(1, D)
    a2 = a[D:, 0].reshape(1, D)
    table3 = word_emb_table.reshape(V // VCH, VCH, D)

    t2d, c = pl.pallas_call(
        _tc_body,
        grid=(V // (8 * VCH),),
        in_specs=[
            pl.BlockSpec((8, VCH, D), lambda i: (i, 0, 0)),
            pl.BlockSpec((1, D), lambda i: (0, 0)),
            pl.BlockSpec((B, D), lambda i: (0, 0)),
            pl.BlockSpec((1, D), lambda i: (0, 0)),
        ],
        out_specs=[
            pl.BlockSpec((8, VCH), lambda i: (i, 0)),
            pl.BlockSpec((B,), lambda i: (0,)),
        ],
        out_shape=[
            jax.ShapeDtypeStruct((V // VCH, VCH), jnp.float32),
            jax.ShapeDtypeStruct((B,), jnp.float32),
        ],
    )(table3, a1, attr_emb, a2)
    t = t2d.reshape(-1)

    words_p = jnp.pad(words.astype(jnp.int32), ((0, 0), (0, LP - L)))

    mesh = plsc.VectorSubcoreMesh(core_axis_name="c", subcore_axis_name="s")
    sc = pl.kernel(
        _sc_body,
        out_type=jax.ShapeDtypeStruct((B, V), jnp.float32),
        mesh=mesh,
        compiler_params=pltpu.CompilerParams(needs_layout_passes=False),
        scratch_types=[
            pltpu.VMEM((RPW, LP), jnp.int32),
            pltpu.VMEM((RPW, LP), jnp.float32),
            pltpu.VMEM((RPW,), jnp.float32),
            pltpu.VMEM((V,), jnp.float32),
            pltpu.SemaphoreType.DMA,
        ],
    )
    return sc(t, c, words_p)
